# baseline (speedup 1.0000x reference)
import jax
import jax.numpy as jnp
from jax import lax
from jax.experimental import pallas as pl
from jax.experimental.pallas import tpu as pltpu
from jax.experimental.pallas import tpu_sc as plsc

N = 10000
E = 320000
NC, NS, L = 2, 16, 16
NW = NC * NS
EW = E // NW
SUBW = 80
ZBLK = 40
NZB = N // ZBLK

H1, H2 = 8, 1
D1 = H1 * L + L
D2 = H2 * L + L

BR = 1000
GRID = N // BR


def _splat(v, h):
  dn = lax.GatherDimensionNumbers(
      offset_dims=(), collapsed_slice_dims=(0,), start_index_map=(0,))
  idx = jnp.full((L, 1), h, dtype=jnp.int32)
  return lax.gather(v, idx, dn, slice_sizes=(1,),
                    mode=lax.GatherScatterMode.PROMISE_IN_BOUNDS)


CHUNK1, CHUNK2 = 80, 400
def _make_edge_sweep_deep(D, H, CHUNK, UNROLL):
  DM = H * L
  NSUB = CHUNK // SUBW
  NCH = EW // CHUNK
  mesh = plsc.VectorSubcoreMesh(core_axis_name="c", subcore_axis_name="s")

  def body(pk_r, tab_r, ad_r, out_r,
           pbuf, sidx3, didx3, rows3, adrows3, acc, sem_g, sem_s, sem_i):
    cid = lax.axis_index("c")
    sid = lax.axis_index("s")
    wid = sid * NC + cid

    def idx_start(kn, t):
      pltpu.async_copy(pk_r.at[pl.ds((wid * NCH + kn) * CHUNK, CHUNK)],
                       pbuf[t], sem_i[t])

    def idx_wait(t):
      pltpu.make_async_copy(pk_r.at[pl.ds(0, CHUNK)], pbuf[t],
                            sem_i[t]).wait()

    def unpack(t):
      for u in range(CHUNK // L):
        p = pbuf[t][pl.ds(u * L, L)]
        r, c = (u * L) // SUBW, (u * L) % SUBW
        sidx3[t][r, pl.ds(c, L)] = jnp.bitwise_and(p, 0xFFFF)
        didx3[t][r, pl.ds(c, L)] = lax.shift_right_logical(p, 16)

    def gather_start(t):
      for j in range(NSUB):
        pltpu.async_copy(tab_r.at[sidx3[t].at[j]],
                         rows3[t].at[pl.ds(j * SUBW, SUBW)], sem_g[t])
        pltpu.async_copy(ad_r.at[didx3[t].at[j]],
                         adrows3[t].at[pl.ds(j * SUBW, SUBW)], sem_g[t])

    def gather_wait(t):
      pltpu.make_async_copy(tab_r.at[pl.ds(0, CHUNK)], rows3[t],
                            sem_g[t]).wait()
      pltpu.make_async_copy(ad_r.at[pl.ds(0, CHUNK)], adrows3[t],
                            sem_g[t]).wait()

    def scatter_start(t):
      for j in range(NSUB):
        pltpu.async_copy(rows3[t].at[pl.ds(j * SUBW, SUBW)],
                         acc.at[didx3[t].at[j]], sem_s, add=True)

    def scatter_wait(t):
      for j in range(NSUB):
        pltpu.make_async_copy(rows3[t].at[pl.ds(j * SUBW, SUBW)],
                              acc.at[didx3[t].at[j]], sem_s).wait()

    def compute(t):
      rows, adrows = rows3[t], adrows3[t]
      @plsc.parallel_loop(0, CHUNK, unroll=UNROLL)
      def _(e):
        av = rows[e, pl.ds(DM, L)]
        dv = adrows[e, pl.ds(0, L)]
        s = av + dv
        s = jnp.where(s > 0, s, 0.2 * s)
        w = jnp.exp(s)
        rows[e, pl.ds(DM, L)] = w
        for h in range(H):
          rows[e, pl.ds(h * L, L)] = rows[e, pl.ds(h * L, L)] * _splat(w, h)

    def z_body(i, _):
      for c in range(D // L):
        rows3[0][i, pl.ds(c * L, L)] = jnp.zeros((L,), jnp.float32)
      return 0
    lax.fori_loop(0, ZBLK, z_body, 0)
    for i in range(NZB // NS + 1):
      q = sid + NS * i
      @pl.when(q < NZB)
      def _():
        pltpu.sync_copy(rows3[0].at[pl.ds(0, ZBLK)],
                        acc.at[pl.ds(q * ZBLK, ZBLK)])
    plsc.subcore_barrier()

    idx_start(0, 0)
    idx_wait(0)
    unpack(0)
    gather_start(0)
    idx_start(1, 1)
    idx_wait(1)
    unpack(1)
    gather_start(1)
    idx_start(2, 2)

    gather_wait(0)
    idx_wait(2)
    unpack(2)
    gather_start(2)
    idx_start(3, 0)
    compute(0)
    scatter_start(0)

    def steady(k, t):
      gather_wait(t)
      t2 = (t + 2) % 3
      scatter_wait(t2)
      idx_wait(t2)
      unpack(t2)
      gather_start(t2)
      idx_start(lax.rem(k + 3, NCH), t)
      compute(t)
      scatter_start(t)

    k0 = 1 + (NCH - 1) % 3
    for k in range(1, k0):
      steady(k, k % 3)

    def tri_body(i, _):
      for d in range(3):
        k = 3 * i + k0 + d
        steady(k, (k0 + d) % 3)
      return 0
    lax.fori_loop(0, (NCH - k0) // 3, tri_body, 0)

    scatter_wait((NCH - 1) % 3)
    gather_wait(NCH % 3)
    gather_wait((NCH + 1) % 3)
    idx_wait((NCH + 2) % 3)

    plsc.subcore_barrier()
    for i in range(NZB // NS + 1):
      q = sid + NS * i
      @pl.when(q < NZB)
      def _():
        pltpu.sync_copy(acc.at[pl.ds(q * ZBLK, ZBLK)],
                        out_r.at[pl.ds(cid * N + q * ZBLK, ZBLK)])

  return pl.kernel(
      body,
      out_type=jax.ShapeDtypeStruct((NC * N, D), jnp.float32),
      mesh=mesh,
      compiler_params=pltpu.CompilerParams(use_tc_tiling_on_sc=False),
      scratch_types=[
          [pltpu.VMEM((CHUNK,), jnp.int32) for _ in range(3)],
          [pltpu.VMEM((NSUB, SUBW), jnp.int32) for _ in range(3)],
          [pltpu.VMEM((NSUB, SUBW), jnp.int32) for _ in range(3)],
          [pltpu.VMEM((CHUNK, D), jnp.float32) for _ in range(3)],
          [pltpu.VMEM((CHUNK, L), jnp.float32) for _ in range(3)],
          pltpu.VMEM_SHARED((N, D), jnp.float32),
          [pltpu.SemaphoreType.DMA for _ in range(3)],
          pltpu.SemaphoreType.DMA,
          [pltpu.SemaphoreType.DMA for _ in range(3)],
      ],
  )


_edge1 = _make_edge_sweep_deep(D1, H1, CHUNK1, 4)
_edge2 = _make_edge_sweep_deep(D2, H2, CHUNK2, 4)


def _tc_layer1(x, W1, As, Ad):
  def body(x_r, w_r, as_r, ad_r, hh_r, adp_r):
    h = jnp.dot(x_r[...], w_r[...], preferred_element_type=jnp.float32)
    a_s = jnp.dot(h, as_r[...], preferred_element_type=jnp.float32)
    a_d = jnp.dot(h, ad_r[...], preferred_element_type=jnp.float32)
    z = jnp.zeros((BR, 8), jnp.float32)
    hh_r[...] = jnp.concatenate([h, a_s, z], axis=1)
    adp_r[...] = jnp.concatenate([a_d, z], axis=1)

  return pl.pallas_call(
      body,
      grid=(GRID,),
      in_specs=[
          pl.BlockSpec((BR, 128), lambda i: (i, 0)),
          pl.BlockSpec((128, 128), lambda i: (0, 0)),
          pl.BlockSpec((128, 8), lambda i: (0, 0)),
          pl.BlockSpec((128, 8), lambda i: (0, 0)),
      ],
      out_specs=[
          pl.BlockSpec((BR, D1), lambda i: (i, 0)),
          pl.BlockSpec((BR, L), lambda i: (i, 0)),
      ],
      out_shape=[
          jax.ShapeDtypeStruct((N, D1), jnp.float32),
          jax.ShapeDtypeStruct((N, L), jnp.float32),
      ],
  )(x, W1, As, Ad)


def _tc_mid(acc1, b1, W2, a2s_t, a2d_t, R16):
  def body(pa_r, pb_r, b1_r, w2_r, s_r, d_r, r16_r, hh_r, adp_r):
    s = pa_r[...] + pb_r[...]
    den = jnp.dot(s[:, 128:144], r16_r[...], preferred_element_type=jnp.float32)
    out1 = s[:, :128] / (den + 1e-16) + b1_r[...]
    hE = jnp.where(out1 > 0, out1, jnp.exp(out1) - 1.0)
    h2 = jnp.dot(hE, w2_r[...], preferred_element_type=jnp.float32)
    a_s = jnp.dot(h2, s_r[...], preferred_element_type=jnp.float32)
    a_d = jnp.dot(h2, d_r[...], preferred_element_type=jnp.float32)
    z = jnp.zeros((BR, 15), jnp.float32)
    hh_r[...] = jnp.concatenate([h2, a_s, z], axis=1)
    adp_r[...] = jnp.concatenate([a_d, z], axis=1)

  return pl.pallas_call(
      body,
      grid=(GRID,),
      in_specs=[
          pl.BlockSpec((BR, D1), lambda i: (i, 0)),
          pl.BlockSpec((BR, D1), lambda i: (i + GRID, 0)),
          pl.BlockSpec((1, 128), lambda i: (0, 0)),
          pl.BlockSpec((128, 16), lambda i: (0, 0)),
          pl.BlockSpec((16, 1), lambda i: (0, 0)),
          pl.BlockSpec((16, 1), lambda i: (0, 0)),
          pl.BlockSpec((16, 128), lambda i: (0, 0)),
      ],
      out_specs=[
          pl.BlockSpec((BR, D2), lambda i: (i, 0)),
          pl.BlockSpec((BR, L), lambda i: (i, 0)),
      ],
      out_shape=[
          jax.ShapeDtypeStruct((N, D2), jnp.float32),
          jax.ShapeDtypeStruct((N, L), jnp.float32),
      ],
  )(acc1, acc1, b1, W2, a2s_t, a2d_t, R16)


def _tc_out(acc2, b2):
  def body(pa_r, pb_r, b2_r, o_r):
    s = pa_r[...] + pb_r[...]
    out = s[:, :16] / (s[:, 16:17] + 1e-16) + b2_r[...]
    m = jnp.max(out, axis=1, keepdims=True)
    ls = jnp.log(jnp.sum(jnp.exp(out - m), axis=1, keepdims=True)) + m
    o_r[...] = out - ls

  return pl.pallas_call(
      body,
      grid=(GRID,),
      in_specs=[
          pl.BlockSpec((BR, D2), lambda i: (i, 0)),
          pl.BlockSpec((BR, D2), lambda i: (i + GRID, 0)),
          pl.BlockSpec((1, 16), lambda i: (0, 0)),
      ],
      out_specs=pl.BlockSpec((BR, 16), lambda i: (i, 0)),
      out_shape=jax.ShapeDtypeStruct((N, 16), jnp.float32),
  )(acc2, acc2, b2)


def kernel(x, edge_index, W1, a_src1, a_dst1, b1, W2, a_src2, a_dst2, b2):
  ei = edge_index.astype(jnp.int32)
  pk = ei[0] | (ei[1] << 16)

  eye8 = jnp.eye(8, dtype=jnp.float32)
  As1 = (eye8[:, None, :] * a_src1[:, :, None]).reshape(128, 8)
  Ad1 = (eye8[:, None, :] * a_dst1[:, :, None]).reshape(128, 8)
  R16 = jnp.concatenate([jnp.repeat(eye8, 16, axis=1),
                         jnp.zeros((8, 128), jnp.float32)], axis=0)

  hh1, adp1 = _tc_layer1(x, W1, As1, Ad1)
  acc1 = _edge1(pk, hh1, adp1)
  hh2, adp2 = _tc_mid(acc1, b1.reshape(1, 128), W2,
                      a_src2.reshape(16, 1), a_dst2.reshape(16, 1), R16)
  acc2 = _edge2(pk, hh2, adp2)
  return _tc_out(acc2, b2.reshape(1, 16))


# speedup vs baseline: 1.0396x; 1.0396x over previous
import jax
import jax.numpy as jnp
from jax import lax
from jax.experimental import pallas as pl
from jax.experimental.pallas import tpu as pltpu
from jax.experimental.pallas import tpu_sc as plsc

N = 10000
E = 320000
NC, NS, L = 2, 16, 16
NW = NC * NS
EW = E // NW
SUBW = 80
ZBLK = 40
NZB = N // ZBLK

H1, H2 = 8, 1
D1 = H1 * L + L
D2 = H2 * L + L

BR = 2000
GRID = N // BR


def _splat(v, h):
  dn = lax.GatherDimensionNumbers(
      offset_dims=(), collapsed_slice_dims=(0,), start_index_map=(0,))
  idx = jnp.full((L, 1), h, dtype=jnp.int32)
  return lax.gather(v, idx, dn, slice_sizes=(1,),
                    mode=lax.GatherScatterMode.PROMISE_IN_BOUNDS)


CHUNK1, CHUNK2 = 80, 400
def _make_edge_sweep_deep(D, H, CHUNK, UNROLL):
  DM = H * L
  NSUB = CHUNK // SUBW
  NCH = EW // CHUNK
  mesh = plsc.VectorSubcoreMesh(core_axis_name="c", subcore_axis_name="s")

  def body(pk_r, tab_r, ad_r, out_r,
           pbuf, sidx3, didx3, rows3, adrows3, acc, sem_g, sem_s, sem_i):
    cid = lax.axis_index("c")
    sid = lax.axis_index("s")
    wid = sid * NC + cid

    def idx_start(kn, t):
      pltpu.async_copy(pk_r.at[pl.ds(wid * NCH + kn, 1)], pbuf[t], sem_i[t])

    def idx_wait(t):
      pltpu.make_async_copy(pk_r.at[pl.ds(0, 1)], pbuf[t], sem_i[t]).wait()

    def unpack(t):
      for u in range(CHUNK // L):
        p = pbuf[t][0, pl.ds(u * L, L)]
        r, c = (u * L) // SUBW, (u * L) % SUBW
        sidx3[t][r, pl.ds(c, L)] = jnp.bitwise_and(p, 0xFFFF)
        didx3[t][r, pl.ds(c, L)] = lax.shift_right_logical(p, 16)

    def gather_start(t):
      for j in range(NSUB):
        pltpu.async_copy(tab_r.at[sidx3[t].at[j]],
                         rows3[t].at[pl.ds(j * SUBW, SUBW)], sem_g[t])
        pltpu.async_copy(ad_r.at[didx3[t].at[j]],
                         adrows3[t].at[pl.ds(j * SUBW, SUBW)], sem_g[t])

    def gather_wait(t):
      pltpu.make_async_copy(tab_r.at[pl.ds(0, CHUNK)], rows3[t],
                            sem_g[t]).wait()
      pltpu.make_async_copy(ad_r.at[pl.ds(0, CHUNK)], adrows3[t],
                            sem_g[t]).wait()

    def scatter_start(t):
      for j in range(NSUB):
        pltpu.async_copy(rows3[t].at[pl.ds(j * SUBW, SUBW)],
                         acc.at[didx3[t].at[j]], sem_s, add=True)

    def scatter_wait(t):
      for j in range(NSUB):
        pltpu.make_async_copy(rows3[t].at[pl.ds(j * SUBW, SUBW)],
                              acc.at[didx3[t].at[j]], sem_s).wait()

    def compute(t):
      rows, adrows = rows3[t], adrows3[t]
      @plsc.parallel_loop(0, CHUNK, unroll=UNROLL)
      def _(e):
        av = rows[e, pl.ds(DM, L)]
        dv = adrows[e, pl.ds(0, L)]
        s = av + dv
        s = jnp.where(s > 0, s, 0.2 * s)
        w = jnp.exp(s)
        rows[e, pl.ds(DM, L)] = w
        for h in range(H):
          rows[e, pl.ds(h * L, L)] = rows[e, pl.ds(h * L, L)] * _splat(w, h)

    def z_body(i, _):
      for c in range(D // L):
        rows3[0][i, pl.ds(c * L, L)] = jnp.zeros((L,), jnp.float32)
      return 0
    lax.fori_loop(0, ZBLK, z_body, 0)
    for i in range(NZB // NS + 1):
      q = sid + NS * i
      @pl.when(q < NZB)
      def _():
        pltpu.sync_copy(rows3[0].at[pl.ds(0, ZBLK)],
                        acc.at[pl.ds(q * ZBLK, ZBLK)])
    plsc.subcore_barrier()

    idx_start(0, 0)
    idx_wait(0)
    unpack(0)
    gather_start(0)
    idx_start(1, 1)
    idx_wait(1)
    unpack(1)
    gather_start(1)
    idx_start(2, 2)

    gather_wait(0)
    idx_wait(2)
    unpack(2)
    gather_start(2)
    idx_start(3, 0)
    compute(0)
    scatter_start(0)

    def steady(k, t):
      gather_wait(t)
      t2 = (t + 2) % 3
      scatter_wait(t2)
      idx_wait(t2)
      unpack(t2)
      gather_start(t2)
      idx_start(lax.rem(k + 3, NCH), t)
      compute(t)
      scatter_start(t)

    k0 = 1 + (NCH - 1) % 3
    for k in range(1, k0):
      steady(k, k % 3)

    def tri_body(i, _):
      for d in range(3):
        k = 3 * i + k0 + d
        steady(k, (k0 + d) % 3)
      return 0
    lax.fori_loop(0, (NCH - k0) // 3, tri_body, 0)

    scatter_wait((NCH - 1) % 3)
    gather_wait(NCH % 3)
    gather_wait((NCH + 1) % 3)
    idx_wait((NCH + 2) % 3)

    plsc.subcore_barrier()
    for i in range(NZB // NS + 1):
      q = sid + NS * i
      @pl.when(q < NZB)
      def _():
        pltpu.sync_copy(acc.at[pl.ds(q * ZBLK, ZBLK)],
                        out_r.at[pl.ds(cid * N + q * ZBLK, ZBLK)])

  return pl.kernel(
      body,
      out_type=jax.ShapeDtypeStruct((NC * N, D), jnp.float32),
      mesh=mesh,
      compiler_params=pltpu.CompilerParams(use_tc_tiling_on_sc=False),
      scratch_types=[
          [pltpu.VMEM((1, CHUNK), jnp.int32) for _ in range(3)],
          [pltpu.VMEM((NSUB, SUBW), jnp.int32) for _ in range(3)],
          [pltpu.VMEM((NSUB, SUBW), jnp.int32) for _ in range(3)],
          [pltpu.VMEM((CHUNK, D), jnp.float32) for _ in range(3)],
          [pltpu.VMEM((CHUNK, L), jnp.float32) for _ in range(3)],
          pltpu.VMEM_SHARED((N, D), jnp.float32),
          [pltpu.SemaphoreType.DMA for _ in range(3)],
          pltpu.SemaphoreType.DMA,
          [pltpu.SemaphoreType.DMA for _ in range(3)],
      ],
  )


_edge1 = _make_edge_sweep_deep(D1, H1, CHUNK1, 2)
_edge2 = _make_edge_sweep_deep(D2, H2, CHUNK2, 4)


def _tc_layer1(x, W1, As, Ad):
  def body(x_r, w_r, as_r, ad_r, hh_r, adp_r):
    h = jnp.dot(x_r[...], w_r[...], preferred_element_type=jnp.float32)
    a_s = jnp.dot(h, as_r[...], preferred_element_type=jnp.float32)
    a_d = jnp.dot(h, ad_r[...], preferred_element_type=jnp.float32)
    z = jnp.zeros((BR, 8), jnp.float32)
    hh_r[...] = jnp.concatenate([h, a_s, z], axis=1)
    adp_r[...] = jnp.concatenate([a_d, z], axis=1)

  return pl.pallas_call(
      body,
      grid=(GRID,),
      in_specs=[
          pl.BlockSpec((BR, 128), lambda i: (i, 0)),
          pl.BlockSpec((128, 128), lambda i: (0, 0)),
          pl.BlockSpec((128, 8), lambda i: (0, 0)),
          pl.BlockSpec((128, 8), lambda i: (0, 0)),
      ],
      out_specs=[
          pl.BlockSpec((BR, D1), lambda i: (i, 0)),
          pl.BlockSpec((BR, L), lambda i: (i, 0)),
      ],
      out_shape=[
          jax.ShapeDtypeStruct((N, D1), jnp.float32),
          jax.ShapeDtypeStruct((N, L), jnp.float32),
      ],
  )(x, W1, As, Ad)


def _tc_mid(acc1, b1, W2, a2s_t, a2d_t, R16):
  def body(pa_r, pb_r, b1_r, w2_r, s_r, d_r, r16_r, hh_r, adp_r):
    s = pa_r[...] + pb_r[...]
    den = jnp.dot(s[:, 128:144], r16_r[...], preferred_element_type=jnp.float32)
    out1 = s[:, :128] / (den + 1e-16) + b1_r[...]
    hE = jnp.where(out1 > 0, out1, jnp.exp(out1) - 1.0)
    h2 = jnp.dot(hE, w2_r[...], preferred_element_type=jnp.float32)
    a_s = jnp.dot(h2, s_r[...], preferred_element_type=jnp.float32)
    a_d = jnp.dot(h2, d_r[...], preferred_element_type=jnp.float32)
    z = jnp.zeros((BR, 15), jnp.float32)
    hh_r[...] = jnp.concatenate([h2, a_s, z], axis=1)
    adp_r[...] = jnp.concatenate([a_d, z], axis=1)

  return pl.pallas_call(
      body,
      grid=(GRID,),
      in_specs=[
          pl.BlockSpec((BR, D1), lambda i: (i, 0)),
          pl.BlockSpec((BR, D1), lambda i: (i + GRID, 0)),
          pl.BlockSpec((1, 128), lambda i: (0, 0)),
          pl.BlockSpec((128, 16), lambda i: (0, 0)),
          pl.BlockSpec((16, 1), lambda i: (0, 0)),
          pl.BlockSpec((16, 1), lambda i: (0, 0)),
          pl.BlockSpec((16, 128), lambda i: (0, 0)),
      ],
      out_specs=[
          pl.BlockSpec((BR, D2), lambda i: (i, 0)),
          pl.BlockSpec((BR, L), lambda i: (i, 0)),
      ],
      out_shape=[
          jax.ShapeDtypeStruct((N, D2), jnp.float32),
          jax.ShapeDtypeStruct((N, L), jnp.float32),
      ],
  )(acc1, acc1, b1, W2, a2s_t, a2d_t, R16)


def _tc_out(acc2, b2):
  def body(pa_r, pb_r, b2_r, o_r):
    s = pa_r[...] + pb_r[...]
    out = s[:, :16] / (s[:, 16:17] + 1e-16) + b2_r[...]
    m = jnp.max(out, axis=1, keepdims=True)
    ls = jnp.log(jnp.sum(jnp.exp(out - m), axis=1, keepdims=True)) + m
    o_r[...] = out - ls

  return pl.pallas_call(
      body,
      grid=(GRID,),
      in_specs=[
          pl.BlockSpec((BR, D2), lambda i: (i, 0)),
          pl.BlockSpec((BR, D2), lambda i: (i + GRID, 0)),
          pl.BlockSpec((1, 16), lambda i: (0, 0)),
      ],
      out_specs=pl.BlockSpec((BR, 16), lambda i: (i, 0)),
      out_shape=jax.ShapeDtypeStruct((N, 16), jnp.float32),
  )(acc2, acc2, b2)


def kernel(x, edge_index, W1, a_src1, a_dst1, b1, W2, a_src2, a_dst2, b2):
  ei = edge_index.astype(jnp.int32)
  pk = ei[0] | (ei[1] << 16)
  pk1 = pk.reshape(E // CHUNK1, CHUNK1)
  pk2 = pk.reshape(E // CHUNK2, CHUNK2)

  eye8 = jnp.eye(8, dtype=jnp.float32)
  As1 = (eye8[:, None, :] * a_src1[:, :, None]).reshape(128, 8)
  Ad1 = (eye8[:, None, :] * a_dst1[:, :, None]).reshape(128, 8)
  R16 = jnp.concatenate([jnp.repeat(eye8, 16, axis=1),
                         jnp.zeros((8, 128), jnp.float32)], axis=0)

  hh1, adp1 = _tc_layer1(x, W1, As1, Ad1)
  acc1 = _edge1(pk1, hh1, adp1)
  hh2, adp2 = _tc_mid(acc1, b1.reshape(1, 128), W2,
                      a_src2.reshape(16, 1), a_dst2.reshape(16, 1), R16)
  acc2 = _edge2(pk2, hh2, adp2)
  return _tc_out(acc2, b2.reshape(1, 16))
